# baseline (speedup 1.0000x reference)
import functools

import jax
import jax.numpy as jnp
from jax import lax
from jax.experimental import pallas as pl
from jax.experimental.pallas import tpu as pltpu
from jax.experimental.pallas import tpu_sc as plsc

B, S, D, V = 4096, 200, 64, 256
ROWS = B * S
IROWS = ROWS // 2

_info = plsc.get_sparse_core_info()
NC, NS = _info.num_cores, _info.num_subcores
NW = NC * NS
ROWS_PER_W = ROWS // NW
CHUNK = 128
HC = CHUNK // 2
NCHUNK = ROWS_PER_W // CHUNK
NBUF = 5
NGROUP = NCHUNK // NBUF
BBLK = 16


def _vt_copy_body(v_ref, o_ref):
    o_ref[...] = v_ref[...]


def _epilogue_body(i_ref, pos_ref, o_ref):
    x = i_ref[...]
    ev = x[:, :D]
    od = x[:, D:]
    y = jnp.stack([ev, od], axis=1)
    y = y.reshape(BBLK, S, D)
    o_ref[...] = y + pos_ref[...][None, :, :]


def _sc_gather_body(vt_hbm, g_hbm, i_hbm, idx_v, bufs, gsem, osem):
    wid = lax.axis_index("s") * NC + lax.axis_index("c")
    chunk_base = wid * NCHUNK
    pltpu.sync_copy(g_hbm.at[pl.ds(chunk_base, NCHUNK), :, :], idx_v)

    def i_slice(j, h):
        return i_hbm.at[pl.ds((chunk_base + j) * HC, HC), pl.ds(h * D, D)]

    def group(t, carry):
        p = lax.rem(t, 2)
        j0 = t * NBUF

        @pl.when(t >= 2)
        def _():
            for b in range(NBUF):
                for h in (0, 1):
                    pltpu.make_async_copy(
                        bufs.at[p, b, h],
                        i_slice(j0 - 2 * NBUF + b, h),
                        osem,
                    ).wait()

        gds = [
            pltpu.async_copy(
                vt_hbm.at[idx_v.at[j0 + b, h]], bufs.at[p, b, h], gsem
            )
            for b in range(NBUF)
            for h in (0, 1)
        ]
        for d in gds:
            d.wait()
        for b in range(NBUF):
            for h in (0, 1):
                pltpu.async_copy(bufs.at[p, b, h], i_slice(j0 + b, h), osem)
        return carry

    lax.fori_loop(0, NGROUP, group, 0)
    for t in (NGROUP - 2, NGROUP - 1):
        for b in range(NBUF):
            for h in (0, 1):
                pltpu.make_async_copy(
                    bufs.at[t % 2, b, h], i_slice(t * NBUF + b, h), osem
                ).wait()


_sc_gather = functools.partial(
    pl.kernel,
    out_type=jax.ShapeDtypeStruct((IROWS, 2 * D), jnp.float32),
    mesh=plsc.VectorSubcoreMesh(core_axis_name="c", subcore_axis_name="s"),
    scratch_types=[
        pltpu.VMEM((NCHUNK, 2, HC), jnp.int32),
        pltpu.VMEM((2, NBUF, 2, HC, D), jnp.float32),
        pltpu.SemaphoreType.DMA,
        pltpu.SemaphoreType.DMA,
    ],
    compiler_params=pltpu.CompilerParams(use_tc_tiling_on_sc=False),
)(_sc_gather_body)


@jax.jit
def kernel(inputs, value_table, pos_table):
    g2 = inputs.reshape(ROWS // CHUNK, HC, 2).transpose(0, 2, 1)
    vt = pl.pallas_call(
        _vt_copy_body,
        out_shape=jax.ShapeDtypeStruct((V, D), jnp.float32),
    )(value_table)
    i_packed = _sc_gather(vt, g2)
    out = pl.pallas_call(
        _epilogue_body,
        grid=(B // BBLK,),
        in_specs=[
            pl.BlockSpec((BBLK * S // 2, 2 * D), lambda i: (i, 0)),
            pl.BlockSpec((S, D), lambda i: (0, 0)),
        ],
        out_specs=pl.BlockSpec((BBLK, S, D), lambda i: (i, 0, 0)),
        out_shape=jax.ShapeDtypeStruct((B, S, D), jnp.float32),
    )(i_packed, pos_table)
    return out


# speedup vs baseline: 2.0803x; 2.0803x over previous
import functools

import jax
import jax.numpy as jnp
from jax import lax
from jax.experimental import pallas as pl
from jax.experimental.pallas import tpu as pltpu
from jax.experimental.pallas import tpu_sc as plsc

B, S, D, V = 4096, 200, 64, 256
ROWS = B * S

_info = plsc.get_sparse_core_info()
NC, NS = _info.num_cores, _info.num_subcores
NW = NC * NS
ROWS_PER_W = ROWS // NW
CHUNK = 100
NCHUNK = ROWS_PER_W // CHUNK
NBUF = 8
NGROUP = NCHUNK // NBUF


def _fused_table_body(pos_ref, val_ref, f_ref):
    f = pos_ref[...][:, None, :] + val_ref[...][None, :, :]
    f_ref[...] = f.reshape(f_ref.shape)


def _gidx_body(inp_ref, g_ref):
    i0 = lax.broadcasted_iota(jnp.int32, g_ref.shape, 0)
    i1 = lax.broadcasted_iota(jnp.int32, g_ref.shape, 1)
    s = lax.rem(i0 * CHUNK + i1, S)
    g_ref[...] = inp_ref[...] + s * V


def _tc_prep(inputs, value_table, pos_table):
    f = pl.pallas_call(
        _fused_table_body,
        grid=(S // 8,),
        in_specs=[
            pl.BlockSpec((8, D), lambda i: (i, 0)),
            pl.BlockSpec((V, D), lambda i: (0, 0)),
        ],
        out_specs=pl.BlockSpec((8 * V, D), lambda i: (i, 0)),
        out_shape=jax.ShapeDtypeStruct((S * V, D), jnp.float32),
    )(pos_table, value_table)
    g = pl.pallas_call(
        _gidx_body,
        out_shape=jax.ShapeDtypeStruct((ROWS // CHUNK, CHUNK), jnp.int32),
    )(inputs.reshape(ROWS // CHUNK, CHUNK))
    return f, g


def _sc_gather_body(f_hbm, g_hbm, out_hbm, idx_v, bufs, gsem, osem):
    wid = lax.axis_index("s") * NC + lax.axis_index("c")
    chunk_base = wid * NCHUNK
    pltpu.sync_copy(g_hbm.at[pl.ds(chunk_base, NCHUNK), :], idx_v)

    def out_slice(j):
        return out_hbm.at[pl.ds((chunk_base + j) * CHUNK, CHUNK), :]

    def group(t, carry):
        p = lax.rem(t, 2)
        j0 = t * NBUF

        @pl.when(t >= 2)
        def _():
            for b in range(NBUF):
                pltpu.make_async_copy(
                    bufs.at[p, b], out_slice(j0 - 2 * NBUF + b), osem
                ).wait()

        gds = [
            pltpu.async_copy(
                f_hbm.at[idx_v.at[j0 + b]], bufs.at[p, b], gsem
            )
            for b in range(NBUF)
        ]
        for d in gds:
            d.wait()
        for b in range(NBUF):
            pltpu.async_copy(bufs.at[p, b], out_slice(j0 + b), osem)
        return carry

    lax.fori_loop(0, NGROUP, group, 0)
    for t in (NGROUP - 2, NGROUP - 1):
        for b in range(NBUF):
            pltpu.make_async_copy(
                bufs.at[t % 2, b], out_slice(t * NBUF + b), osem
            ).wait()


_sc_gather = functools.partial(
    pl.kernel,
    out_type=jax.ShapeDtypeStruct((ROWS, D), jnp.float32),
    mesh=plsc.VectorSubcoreMesh(core_axis_name="c", subcore_axis_name="s"),
    scratch_types=[
        pltpu.VMEM((NCHUNK, CHUNK), jnp.int32),
        pltpu.VMEM((2, NBUF, CHUNK, D), jnp.float32),
        pltpu.SemaphoreType.DMA,
        pltpu.SemaphoreType.DMA,
    ],
    compiler_params=pltpu.CompilerParams(use_tc_tiling_on_sc=False),
)(_sc_gather_body)


@jax.jit
def kernel(inputs, value_table, pos_table):
    f, g = _tc_prep(inputs, value_table, pos_table)
    out = _sc_gather(f, g)
    return out.reshape(B, S, D)
